# baseline (speedup 1.0000x reference)
import functools

import jax
import jax.numpy as jnp
import numpy as np
from jax import lax
from jax.experimental import pallas as pl
from jax.experimental.pallas import tpu as pltpu


def _sph2(pts):
    n = pts / jnp.linalg.norm(pts, axis=-1, keepdims=True)
    x, y, z = n[..., 0], n[..., 1], n[..., 2]
    s15 = 15.0 ** 0.5
    s5 = 5.0 ** 0.5
    sh = jnp.stack([
        s15 * x * z,
        s15 * x * y,
        s5 * (y ** 2 - 0.5 * (x ** 2 + z ** 2)),
        s15 * y * z,
        (s15 / 2.0) * (z ** 2 - x ** 2),
    ], axis=-1)
    return sh / (4.0 * np.pi) ** 0.5


def _mlp_body(nblk, A, S, D,
              x_ref, wt_ref, bcat_ref, amat_ref, w2sel_ref, sel_ref,
              biascnt_ref, batch_ref, acc_ref):
    i = pl.program_id(0)

    @pl.when(i == 0)
    def _():
        acc_ref[...] = jnp.zeros_like(acc_ref)

    h = lax.dot_general(x_ref[...], wt_ref[...],
                        (((1,), (0,)), ((), ())),
                        preferred_element_type=jnp.float32)
    z = h + bcat_ref[...]
    z = z * jax.nn.sigmoid(z)

    for a in range(A):
        z_a = z[a * S:(a + 1) * S, :]
        p_a = lax.dot_general(amat_ref[...], z_a,
                              (((0,), (0,)), ((), ())),
                              preferred_element_type=jnp.float32)
        out8 = jnp.sum(p_a * w2sel_ref[...], axis=1, keepdims=True)
        row = lax.dot_general(out8, sel_ref[...],
                              (((0,), (0,)), ((), ())),
                              preferred_element_type=jnp.float32)
        row = row + biascnt_ref[...]
        idx = batch_ref[i * A + a]
        acc_ref[pl.ds(idx, 1), :] += row

    @pl.when(i == nblk - 1)
    def _():
        acc = acc_ref[...]
        cnt = jnp.maximum(acc[:, 6:7], 1.0)
        acc_ref[...] = acc / cnt


def kernel(x_pointwise, sphere_points, batch, natoms,
           W1s, b1s, W2s, b2s, W1i, b1i, W2i, b2i):
    N, S, D = x_pointwise.shape
    G = natoms.shape[0]
    A = 8
    R = A * S
    nblk = N // A

    sph = _sph2(sphere_points)
    xf = x_pointwise.reshape(N * S, D)

    wt = jnp.concatenate([W1s.T, W1i.T], axis=1)
    bcat = jnp.concatenate([b1s, b1i]).reshape(1, 2 * D)

    amat = jnp.concatenate([
        jnp.full((S, 1), 1.0 / S, jnp.float32),
        sph / S,
        jnp.zeros((S, 2), jnp.float32),
    ], axis=1)

    w2s = W2s.reshape(D)
    w2i = W2i.reshape(D)
    zd = jnp.zeros((D,), jnp.float32)
    w2sel = jnp.stack([
        jnp.concatenate([w2s, zd]),
    ] + [jnp.concatenate([zd, w2i])] * 5 + [jnp.zeros((2 * D,), jnp.float32)] * 2,
        axis=0)

    sel = jnp.zeros((8, 16), jnp.float32)
    sel = sel.at[0, 0].set(1.0)
    for k in range(1, 6):
        sel = sel.at[k, k].set(1.0)

    msph = jnp.mean(sph, axis=0)
    biascnt = jnp.zeros((16,), jnp.float32)
    biascnt = biascnt.at[0].set(b2s[0])
    biascnt = biascnt.at[1:6].set(b2i[0] * msph)
    biascnt = biascnt.at[6].set(1.0)
    biascnt = biascnt.reshape(1, 16)

    batch_i32 = batch.astype(jnp.int32)

    acc = pl.pallas_call(
        functools.partial(_mlp_body, nblk, A, S, D),
        grid=(nblk,),
        in_specs=[
            pl.BlockSpec((R, D), lambda i: (i, 0)),
            pl.BlockSpec((D, 2 * D), lambda i: (0, 0)),
            pl.BlockSpec((1, 2 * D), lambda i: (0, 0)),
            pl.BlockSpec((S, 8), lambda i: (0, 0)),
            pl.BlockSpec((8, 2 * D), lambda i: (0, 0)),
            pl.BlockSpec((8, 16), lambda i: (0, 0)),
            pl.BlockSpec((1, 16), lambda i: (0, 0)),
            pl.BlockSpec(memory_space=pltpu.SMEM),
        ],
        out_specs=pl.BlockSpec((G, 16), lambda i: (0, 0)),
        out_shape=jax.ShapeDtypeStruct((G, 16), jnp.float32),
        compiler_params=pltpu.CompilerParams(
            dimension_semantics=("arbitrary",),
        ),
    )(xf, wt, bcat, amat, w2sel, sel, biascnt, batch_i32)

    return acc[:, 0], acc[:, 1:6]


# speedup vs baseline: 2.4352x; 2.4352x over previous
import functools

import jax
import jax.numpy as jnp
import numpy as np
from jax import lax
from jax.experimental import pallas as pl
from jax.experimental.pallas import tpu as pltpu
from jax.experimental.pallas import tpu_sc as plsc


def _sph2(pts):
    n = pts / jnp.linalg.norm(pts, axis=-1, keepdims=True)
    x, y, z = n[..., 0], n[..., 1], n[..., 2]
    s15 = 15.0 ** 0.5
    s5 = 5.0 ** 0.5
    sh = jnp.stack([
        s15 * x * z,
        s15 * x * y,
        s5 * (y ** 2 - 0.5 * (x ** 2 + z ** 2)),
        s15 * y * z,
        (s15 / 2.0) * (z ** 2 - x ** 2),
    ], axis=-1)
    return sh / (4.0 * np.pi) ** 0.5


def _mlp_body(A, S,
              x_ref, wt_ref, bcat_ref, mat_ref, mbt_ref,
              w2s_ref, w2i_ref, bs_ref, bi_ref,
              scal_ref, irr_ref):
    h = lax.dot_general(x_ref[...], wt_ref[...],
                        (((1,), (0,)), ((), ())),
                        preferred_element_type=jnp.float32)
    z = h + bcat_ref[...]
    z = z * jax.nn.sigmoid(z)

    t_s = lax.dot_general(mat_ref[...], z, (((1,), (0,)), ((), ())),
                          preferred_element_type=jnp.float32)
    t_i = lax.dot_general(mbt_ref[...], z, (((1,), (0,)), ((), ())),
                          preferred_element_type=jnp.float32)

    scal_ref[...] = jnp.sum(t_s * w2s_ref[...], axis=1, keepdims=True) + bs_ref[...]
    irr_ref[...] = jnp.sum(t_i * w2i_ref[...], axis=1, keepdims=True) + bi_ref[...]


def _mlp_fold(x_pointwise, sph, W1s, b1s, W2s, b2s, W1i, b1i, W2i, b2i, A):
    N, S, D = x_pointwise.shape
    R = A * S
    nblk = N // A

    xf = x_pointwise.reshape(N * S, D)
    wt = jnp.concatenate([W1s.T, W1i.T], axis=1)
    bcat = jnp.concatenate([b1s, b1i]).reshape(1, 2 * D)

    mat = jnp.kron(jnp.eye(A, dtype=jnp.float32),
                   jnp.full((1, S), 1.0 / S, jnp.float32))
    mbt = jnp.kron(jnp.eye(A, dtype=jnp.float32), sph.T / S)

    zd = jnp.zeros((D,), jnp.float32)
    w2srow = jnp.concatenate([W2s.reshape(D), zd]).reshape(1, 2 * D)
    w2irow = jnp.concatenate([zd, W2i.reshape(D)]).reshape(1, 2 * D)

    msph = jnp.mean(sph, axis=0)
    bs_col = jnp.full((A, 1), b2s[0], jnp.float32)
    bi_col = jnp.tile(b2i[0] * msph, A).reshape(5 * A, 1)

    scal, irr = pl.pallas_call(
        functools.partial(_mlp_body, A, S),
        grid=(nblk,),
        in_specs=[
            pl.BlockSpec((R, D), lambda i: (i, 0)),
            pl.BlockSpec((D, 2 * D), lambda i: (0, 0)),
            pl.BlockSpec((1, 2 * D), lambda i: (0, 0)),
            pl.BlockSpec((A, R), lambda i: (0, 0)),
            pl.BlockSpec((5 * A, R), lambda i: (0, 0)),
            pl.BlockSpec((1, 2 * D), lambda i: (0, 0)),
            pl.BlockSpec((1, 2 * D), lambda i: (0, 0)),
            pl.BlockSpec((A, 1), lambda i: (0, 0)),
            pl.BlockSpec((5 * A, 1), lambda i: (0, 0)),
        ],
        out_specs=[
            pl.BlockSpec((A, 1), lambda i: (i, 0)),
            pl.BlockSpec((5 * A, 1), lambda i: (i, 0)),
        ],
        out_shape=[
            jax.ShapeDtypeStruct((N, 1), jnp.float32),
            jax.ShapeDtypeStruct((N * 5, 1), jnp.float32),
        ],
        compiler_params=pltpu.CompilerParams(
            dimension_semantics=("arbitrary",),
        ),
    )(xf, wt, bcat, mat, mbt, w2srow, w2irow, bs_col, bi_col)
    return scal, irr.reshape(N, 5)


def _segment_mean(p16, batch_t, G):
    N = p16.shape[0]
    NSUB = 16
    CH = N // NSUB
    NROW = CH // 128
    GP = G // NSUB

    mesh = plsc.VectorSubcoreMesh(core_axis_name="c", subcore_axis_name="s")
    zeros128 = jnp.zeros((G, 128), jnp.float32)

    @functools.partial(
        pl.kernel, mesh=mesh,
        out_type=jax.ShapeDtypeStruct((G, 128), jnp.float32),
        scratch_types=[
            pltpu.VMEM((NROW, 128), jnp.int32),
            pltpu.VMEM((CH, 128), jnp.float32),
            pltpu.VMEM((GP, 128), jnp.float32),
            pltpu.VMEM_SHARED((G, 128), jnp.float32),
        ],
    )
    def seg_kernel(p_hbm, bt_hbm, z_hbm, out_hbm, idx_v, rows_v, fin_v, acc_sh):
        c = lax.axis_index("c")
        s = lax.axis_index("s")
        on0 = c == 0

        @pl.when(on0)
        def _():
            pltpu.sync_copy(z_hbm.at[pl.ds(s * GP, GP)],
                            acc_sh.at[pl.ds(s * GP, GP)])

        plsc.subcore_barrier()

        @pl.when(on0)
        def _():
            pltpu.sync_copy(bt_hbm.at[pl.ds(s * NROW, NROW)], idx_v)
            pltpu.sync_copy(p_hbm.at[pl.ds(s * CH, CH)], rows_v)
            for q in range(NROW):
                pltpu.sync_copy(rows_v.at[pl.ds(q * 128, 128)],
                                acc_sh.at[idx_v.at[q]], add=True)

        plsc.subcore_barrier()

        @pl.when(on0)
        def _():
            pltpu.sync_copy(acc_sh.at[pl.ds(s * GP, GP)], fin_v)
            lane6 = jnp.full((16, 1), 6, jnp.int32)
            dnums = lax.GatherDimensionNumbers(
                offset_dims=(), collapsed_slice_dims=(0,),
                start_index_map=(0,))
            for r in range(GP):
                v = fin_v[r, 0:16]
                cnt = lax.gather(v, lane6, dnums, (1,),
                                 mode=lax.GatherScatterMode.PROMISE_IN_BOUNDS)
                fin_v[r, 0:16] = v / jnp.maximum(cnt, 1.0)
            pltpu.sync_copy(fin_v, out_hbm.at[pl.ds(s * GP, GP)])

    return seg_kernel(p16, batch_t, zeros128)


def kernel(x_pointwise, sphere_points, batch, natoms,
           W1s, b1s, W2s, b2s, W1i, b1i, W2i, b2i):
    N, S, D = x_pointwise.shape
    G = natoms.shape[0]
    A = 8

    sph = _sph2(sphere_points)
    scal, irr = _mlp_fold(x_pointwise, sph,
                          W1s, b1s, W2s, b2s, W1i, b1i, W2i, b2i, A)

    p16 = jnp.concatenate(
        [scal, irr, jnp.ones((N, 1), jnp.float32),
         jnp.zeros((N, 121), jnp.float32)], axis=1)
    batch_t = batch.astype(jnp.int32).reshape(N // 128, 128)

    acc = _segment_mean(p16, batch_t, G)
    return acc[:, 0], acc[:, 1:6]


# speedup vs baseline: 3.7763x; 1.5507x over previous
import functools

import jax
import jax.numpy as jnp
import numpy as np
from jax import lax
from jax.experimental import pallas as pl
from jax.experimental.pallas import tpu as pltpu
from jax.experimental.pallas import tpu_sc as plsc


def _sph2(pts):
    n = pts / jnp.linalg.norm(pts, axis=-1, keepdims=True)
    x, y, z = n[..., 0], n[..., 1], n[..., 2]
    s15 = 15.0 ** 0.5
    s5 = 5.0 ** 0.5
    sh = jnp.stack([
        s15 * x * z,
        s15 * x * y,
        s5 * (y ** 2 - 0.5 * (x ** 2 + z ** 2)),
        s15 * y * z,
        (s15 / 2.0) * (z ** 2 - x ** 2),
    ], axis=-1)
    return sh / (4.0 * np.pi) ** 0.5


def _mlp_body(A, S,
              x_ref, wt_ref, bcat_ref, mat_ref, mbt_ref,
              w2s_ref, w2i_ref, bs_ref, bi_ref,
              scal_ref, irr_ref):
    h = lax.dot_general(x_ref[...].astype(jnp.bfloat16), wt_ref[...],
                        (((1,), (0,)), ((), ())),
                        preferred_element_type=jnp.float32)
    z = h + bcat_ref[...]
    z = (z * jax.nn.sigmoid(z)).astype(jnp.bfloat16)

    t_s = lax.dot_general(mat_ref[...], z, (((1,), (0,)), ((), ())),
                          preferred_element_type=jnp.float32)
    t_i = lax.dot_general(mbt_ref[...], z, (((1,), (0,)), ((), ())),
                          preferred_element_type=jnp.float32)

    scal_ref[...] = jnp.sum(t_s * w2s_ref[...], axis=1, keepdims=True) + bs_ref[...]
    irr_ref[...] = jnp.sum(t_i * w2i_ref[...], axis=1, keepdims=True) + bi_ref[...]


def _mlp_fold(x_pointwise, sph, W1s, b1s, W2s, b2s, W1i, b1i, W2i, b2i, A):
    N, S, D = x_pointwise.shape
    R = A * S
    nblk = N // A

    xf = x_pointwise.reshape(N * S, D)
    wt = jnp.concatenate([W1s.T, W1i.T], axis=1).astype(jnp.bfloat16)
    bcat = jnp.concatenate([b1s, b1i]).reshape(1, 2 * D)

    mat = jnp.kron(jnp.eye(A, dtype=jnp.float32),
                   jnp.full((1, S), 1.0 / S, jnp.float32)).astype(jnp.bfloat16)
    mbt = jnp.kron(jnp.eye(A, dtype=jnp.float32), sph.T / S).astype(jnp.bfloat16)

    zd = jnp.zeros((D,), jnp.float32)
    w2srow = jnp.concatenate([W2s.reshape(D), zd]).reshape(1, 2 * D)
    w2irow = jnp.concatenate([zd, W2i.reshape(D)]).reshape(1, 2 * D)

    msph = jnp.mean(sph, axis=0)
    bs_col = jnp.full((A, 1), b2s[0], jnp.float32)
    bi_col = jnp.tile(b2i[0] * msph, A).reshape(5 * A, 1)

    scal, irr = pl.pallas_call(
        functools.partial(_mlp_body, A, S),
        grid=(nblk,),
        in_specs=[
            pl.BlockSpec((R, D), lambda i: (i, 0)),
            pl.BlockSpec((D, 2 * D), lambda i: (0, 0)),
            pl.BlockSpec((1, 2 * D), lambda i: (0, 0)),
            pl.BlockSpec((A, R), lambda i: (0, 0)),
            pl.BlockSpec((5 * A, R), lambda i: (0, 0)),
            pl.BlockSpec((1, 2 * D), lambda i: (0, 0)),
            pl.BlockSpec((1, 2 * D), lambda i: (0, 0)),
            pl.BlockSpec((A, 1), lambda i: (0, 0)),
            pl.BlockSpec((5 * A, 1), lambda i: (0, 0)),
        ],
        out_specs=[
            pl.BlockSpec((A, 1), lambda i: (i, 0)),
            pl.BlockSpec((5 * A, 1), lambda i: (i, 0)),
        ],
        out_shape=[
            jax.ShapeDtypeStruct((N, 1), jnp.float32),
            jax.ShapeDtypeStruct((N * 5, 1), jnp.float32),
        ],
        compiler_params=pltpu.CompilerParams(
            dimension_semantics=("arbitrary",),
        ),
    )(xf, wt, bcat, mat, mbt, w2srow, w2irow, bs_col, bi_col)
    return scal, irr.reshape(N, 5)


def _segment_mean(p16, batch_t, G):
    N = p16.shape[0]
    NSUB = 16
    CH = N // NSUB
    NROW = CH // 128
    GP = G // NSUB

    mesh = plsc.VectorSubcoreMesh(core_axis_name="c", subcore_axis_name="s")
    zeros128 = jnp.zeros((G, 128), jnp.float32)

    @functools.partial(
        pl.kernel, mesh=mesh,
        out_type=jax.ShapeDtypeStruct((G, 128), jnp.float32),
        scratch_types=[
            pltpu.VMEM((NROW, 128), jnp.int32),
            pltpu.VMEM((CH, 128), jnp.float32),
            pltpu.VMEM((GP, 128), jnp.float32),
            pltpu.VMEM_SHARED((G, 128), jnp.float32),
        ],
    )
    def seg_kernel(p_hbm, bt_hbm, z_hbm, out_hbm, idx_v, rows_v, fin_v, acc_sh):
        c = lax.axis_index("c")
        s = lax.axis_index("s")
        on0 = c == 0

        @pl.when(on0)
        def _():
            pltpu.sync_copy(z_hbm.at[pl.ds(s * GP, GP)],
                            acc_sh.at[pl.ds(s * GP, GP)])

        plsc.subcore_barrier()

        @pl.when(on0)
        def _():
            pltpu.sync_copy(bt_hbm.at[pl.ds(s * NROW, NROW)], idx_v)
            pltpu.sync_copy(p_hbm.at[pl.ds(s * CH, CH)], rows_v)
            for q in range(NROW):
                pltpu.sync_copy(rows_v.at[pl.ds(q * 128, 128)],
                                acc_sh.at[idx_v.at[q]], add=True)

        plsc.subcore_barrier()

        @pl.when(on0)
        def _():
            pltpu.sync_copy(acc_sh.at[pl.ds(s * GP, GP)], fin_v)
            lane6 = jnp.full((16, 1), 6, jnp.int32)
            dnums = lax.GatherDimensionNumbers(
                offset_dims=(), collapsed_slice_dims=(0,),
                start_index_map=(0,))
            for r in range(GP):
                v = fin_v[r, 0:16]
                cnt = lax.gather(v, lane6, dnums, (1,),
                                 mode=lax.GatherScatterMode.PROMISE_IN_BOUNDS)
                fin_v[r, 0:16] = v / jnp.maximum(cnt, 1.0)
            pltpu.sync_copy(fin_v, out_hbm.at[pl.ds(s * GP, GP)])

    return seg_kernel(p16, batch_t, zeros128)


def kernel(x_pointwise, sphere_points, batch, natoms,
           W1s, b1s, W2s, b2s, W1i, b1i, W2i, b2i):
    N, S, D = x_pointwise.shape
    G = natoms.shape[0]
    A = 32

    sph = _sph2(sphere_points)
    scal, irr = _mlp_fold(x_pointwise, sph,
                          W1s, b1s, W2s, b2s, W1i, b1i, W2i, b2i, A)

    p16 = jnp.concatenate(
        [scal, irr, jnp.ones((N, 1), jnp.float32),
         jnp.zeros((N, 121), jnp.float32)], axis=1)
    batch_t = batch.astype(jnp.int32).reshape(N // 128, 128)

    acc = _segment_mean(p16, batch_t, G)
    return acc[:, 0], acc[:, 1:6]
